# baseline (speedup 1.0000x reference)
import functools

import jax
import jax.numpy as jnp
from jax import lax
from jax.experimental import pallas as pl
from jax.experimental.pallas import tpu as pltpu
from jax.experimental.pallas import tpu_sc as plsc

_CHUNK = 128


@functools.lru_cache(maxsize=None)
def _build(B, V, D):
    info = plsc.get_sparse_core_info()
    NC, NS = info.num_cores, info.num_subcores
    NW = NC * NS
    n_chunks = B // _CHUNK
    cpw = n_chunks // NW

    mesh = plsc.VectorSubcoreMesh(core_axis_name="c", subcore_axis_name="s")

    @functools.partial(
        pl.kernel,
        mesh=mesh,
        out_type=jax.ShapeDtypeStruct((n_chunks, _CHUNK, D), jnp.float32),
        compiler_params=pltpu.CompilerParams(use_tc_tiling_on_sc=False),
        scratch_types=[
            pltpu.VMEM((cpw, _CHUNK), jnp.int32),
            pltpu.VMEM((cpw, _CHUNK, D), jnp.float32),
            pltpu.SemaphoreType.DMA,
        ],
    )
    def gather_kernel(table_hbm, idx_hbm, out_hbm, idx_v, rows_v, sem):
        wid = lax.axis_index("s") * NC + lax.axis_index("c")
        base = wid * cpw
        pltpu.sync_copy(idx_hbm.at[pl.ds(base, cpw)], idx_v)
        copies = [
            pltpu.async_copy(table_hbm.at[idx_v.at[j]], rows_v.at[j], sem)
            for j in range(cpw)
        ]
        for c in copies:
            c.wait()
        pltpu.sync_copy(rows_v, out_hbm.at[pl.ds(base, cpw)])

    return gather_kernel, n_chunks


def kernel(x, style_shift_weight):
    B, = x.shape
    V, D = style_shift_weight.shape
    gather_kernel, n_chunks = _build(B, V, D)
    idx = x.astype(jnp.int32).reshape(n_chunks, _CHUNK)
    out = gather_kernel(style_shift_weight, idx)
    return out.reshape(B, D)


# speedup vs baseline: 3.5362x; 3.5362x over previous
import functools

import jax
import jax.numpy as jnp
from jax import lax
from jax.experimental import pallas as pl
from jax.experimental.pallas import tpu as pltpu
from jax.experimental.pallas import tpu_sc as plsc

_LANES = 128
_GRP = 16


@functools.lru_cache(maxsize=None)
def _build(B, V, D):
    info = plsc.get_sparse_core_info()
    NC, NS = info.num_cores, info.num_subcores
    NW = NC * NS
    bpw = B // NW

    mesh = plsc.VectorSubcoreMesh(core_axis_name="c", subcore_axis_name="s")

    @functools.partial(
        pl.kernel,
        mesh=mesh,
        out_type=jax.ShapeDtypeStruct((D, B), jnp.float32),
        compiler_params=pltpu.CompilerParams(needs_layout_passes=False),
        scratch_types=[
            pltpu.VMEM((bpw,), jnp.int32),
            pltpu.VMEM((_GRP, D, _LANES), jnp.float32),
            pltpu.VMEM((D, bpw), jnp.float32),
            pltpu.SemaphoreType.DMA,
        ],
    )
    def gather_kernel(table_hbm, idx_hbm, out_hbm, idx_v, win_v, vals_v, sem):
        wid = lax.axis_index("s") * NC + lax.axis_index("c")
        base = wid * bpw
        pltpu.sync_copy(idx_hbm.at[pl.ds(base, bpw)], idx_v)

        iota16 = lax.iota(jnp.int32, 16)
        zeros16 = iota16 * 0
        rows = [iota16 + 16 * g for g in range(D // 16)]

        def wait_one():
            pltpu.make_async_copy(
                table_hbm.at[:, pl.ds(0, _LANES)],
                win_v.at[0],
                sem,
            ).wait()

        @pl.loop(0, bpw // _GRP)
        def _(g):
            j0 = g * _GRP
            vec = idx_v[pl.ds(j0, _GRP)]
            cols = [vec[k] for k in range(_GRP)]
            for k in range(_GRP):
                w = lax.shift_right_logical(cols[k], 7)
                pltpu.async_copy(
                    table_hbm.at[:, pl.ds(w * _LANES, _LANES)],
                    win_v.at[k],
                    sem,
                )
            for k in range(_GRP):
                wait_one()
            for k in range(_GRP):
                lane = zeros16 + (cols[k] & (_LANES - 1))
                col_j = zeros16 + (j0 + k)
                for r in rows:
                    vals = plsc.load_gather(win_v.at[k], [r, lane])
                    plsc.store_scatter(vals_v, [r, col_j], vals)

        pltpu.sync_copy(vals_v, out_hbm.at[:, pl.ds(base, bpw)])

    return gather_kernel


def kernel(x, style_shift_weight):
    B, = x.shape
    V, D = style_shift_weight.shape
    gather_kernel = _build(B, V, D)
    idx = x.astype(jnp.int32)
    out_t = gather_kernel(style_shift_weight.T, idx)
    return out_t.T
